# baseline (speedup 1.0000x reference)
import jax
import jax.numpy as jnp
from jax import lax
from jax.experimental import pallas as pl
from jax.experimental.pallas import tpu as pltpu
from jax.experimental.pallas import tpu_sc as plsc

_B, _N, _D, _E = 4, 10000, 128, 320000
_NC, _NS = 2, 16
_NW = _NC * _NS
_EPW = _E // _NW
_C = 128
_R = 3
_NFULL = _EPW // _C
_NBLK = _NFULL // _R
_TAIL = _EPW - _C


def _gather_body(table, idx_i, idx_j, out_i, out_j,
                 idx_i_v, idx_j_v, rows_i, rows_j, *sems):
    gs = (sems[0:_R], sems[_R:2 * _R])
    ss = (sems[2 * _R:3 * _R], sems[3 * _R:4 * _R])
    idx_v = (idx_i_v, idx_j_v)
    rows = (rows_i, rows_j)
    outs = (out_i, out_j)

    wid = lax.axis_index("s") * _NC + lax.axis_index("c")
    ebase = wid * _EPW
    pltpu.sync_copy(idx_i.at[wid], idx_i_v)
    pltpu.sync_copy(idx_j.at[wid], idx_j_v)

    def gather_cp(ep, b, off):
        return pltpu.make_async_copy(
            table.at[idx_v[ep].at[pl.ds(off, _C)]], rows[ep].at[b],
            gs[ep][b])

    def store_cp(ep, b, off):
        return pltpu.make_async_copy(
            rows[ep].at[b], outs[ep].at[pl.ds(ebase + off, _C)], ss[ep][b])

    for b in range(_R):
        for ep in range(2):
            gather_cp(ep, b, b * _C).start()

    def block(t, carry):
        cps = []
        for b in range(_R):
            off = (t * _R + b) * _C
            for ep in range(2):
                gather_cp(ep, b, off).wait()
                cp = store_cp(ep, b, off)
                cp.start()
                cps.append(cp)
        for b in range(_R):
            for ep in range(2):
                cps[2 * b + ep].wait()

            @pl.when(t < _NBLK - 1)
            def _():
                off2 = ((t + 1) * _R + b) * _C
                for ep in range(2):
                    gather_cp(ep, b, off2).start()
        return carry

    lax.fori_loop(0, _NBLK, block, 0)

    for ep in range(2):
        gather_cp(ep, 0, _TAIL).start()
    for ep in range(2):
        gather_cp(ep, 0, _TAIL).wait()
        store_cp(ep, 0, _TAIL).start()
    for ep in range(2):
        store_cp(ep, 0, _TAIL).wait()


@jax.jit
def _gather(table, idx_i, idx_j):
    mesh = plsc.VectorSubcoreMesh(
        core_axis_name="c", subcore_axis_name="s",
        num_cores=_NC, num_subcores=_NS,
    )
    return pl.kernel(
        _gather_body,
        out_type=(
            jax.ShapeDtypeStruct((_E, _D), jnp.float32),
            jax.ShapeDtypeStruct((_E, _D), jnp.float32),
        ),
        mesh=mesh,
        scratch_types=[
            pltpu.VMEM((_EPW,), jnp.int32),
            pltpu.VMEM((_EPW,), jnp.int32),
            pltpu.VMEM((_R, _C, _D), jnp.float32),
            pltpu.VMEM((_R, _C, _D), jnp.float32),
        ] + [pltpu.SemaphoreType.DMA] * (4 * _R),
    )(table, idx_i, idx_j)


def kernel(inputs, selected_edges):
    table = inputs.reshape(_B * _N, _D)
    idx_i = selected_edges[:, 4].reshape(_NW, _EPW)
    idx_j = selected_edges[:, 5].reshape(_NW, _EPW)
    return _gather(table, idx_i, idx_j)


# speedup vs baseline: 1.0163x; 1.0163x over previous
import jax
import jax.numpy as jnp
from jax import lax
from jax.experimental import pallas as pl
from jax.experimental.pallas import tpu as pltpu
from jax.experimental.pallas import tpu_sc as plsc

_B, _N, _D, _E = 4, 10000, 128, 320000
_NC, _NS = 2, 16
_NW = _NC * _NS
_EPW = _E // _NW
_C = 64
_R = 6
_NFULL = _EPW // _C
_NBLK = _NFULL // _R
_TAIL = _EPW - _C


def _gather_body(table, idx_i, idx_j, out_i, out_j,
                 idx_i_v, idx_j_v, rows_i, rows_j, *sems):
    gs = (sems[0:_R], sems[_R:2 * _R])
    ss = (sems[2 * _R:3 * _R], sems[3 * _R:4 * _R])
    idx_v = (idx_i_v, idx_j_v)
    rows = (rows_i, rows_j)
    outs = (out_i, out_j)

    wid = lax.axis_index("s") * _NC + lax.axis_index("c")
    ebase = wid * _EPW
    pltpu.sync_copy(idx_i.at[wid], idx_i_v)
    pltpu.sync_copy(idx_j.at[wid], idx_j_v)

    def gather_cp(ep, b, off):
        return pltpu.make_async_copy(
            table.at[idx_v[ep].at[pl.ds(off, _C)]], rows[ep].at[b],
            gs[ep][b])

    def store_cp(ep, b, off):
        return pltpu.make_async_copy(
            rows[ep].at[b], outs[ep].at[pl.ds(ebase + off, _C)], ss[ep][b])

    for b in range(_R):
        for ep in range(2):
            gather_cp(ep, b, b * _C).start()

    def block(t, carry):
        cps = []
        for b in range(_R):
            off = (t * _R + b) * _C
            for ep in range(2):
                gather_cp(ep, b, off).wait()
                cp = store_cp(ep, b, off)
                cp.start()
                cps.append(cp)
        for b in range(_R):
            for ep in range(2):
                cps[2 * b + ep].wait()

            @pl.when(t < _NBLK - 1)
            def _():
                off2 = ((t + 1) * _R + b) * _C
                for ep in range(2):
                    gather_cp(ep, b, off2).start()
        return carry

    lax.fori_loop(0, _NBLK, block, 0)

    for ep in range(2):
        gather_cp(ep, 0, _TAIL).start()
    for ep in range(2):
        gather_cp(ep, 0, _TAIL).wait()
        store_cp(ep, 0, _TAIL).start()
    for ep in range(2):
        store_cp(ep, 0, _TAIL).wait()


@jax.jit
def _gather(table, idx_i, idx_j):
    mesh = plsc.VectorSubcoreMesh(
        core_axis_name="c", subcore_axis_name="s",
        num_cores=_NC, num_subcores=_NS,
    )
    return pl.kernel(
        _gather_body,
        out_type=(
            jax.ShapeDtypeStruct((_E, _D), jnp.float32),
            jax.ShapeDtypeStruct((_E, _D), jnp.float32),
        ),
        mesh=mesh,
        scratch_types=[
            pltpu.VMEM((_EPW,), jnp.int32),
            pltpu.VMEM((_EPW,), jnp.int32),
            pltpu.VMEM((_R, _C, _D), jnp.float32),
            pltpu.VMEM((_R, _C, _D), jnp.float32),
        ] + [pltpu.SemaphoreType.DMA] * (4 * _R),
    )(table, idx_i, idx_j)


def kernel(inputs, selected_edges):
    table = inputs.reshape(_B * _N, _D)
    idx_i = selected_edges[:, 4].reshape(_NW, _EPW)
    idx_j = selected_edges[:, 5].reshape(_NW, _EPW)
    return _gather(table, idx_i, idx_j)
